# baseline (speedup 1.0000x reference)
import functools

import jax
import jax.numpy as jnp
from jax import lax
from jax.experimental import pallas as pl
from jax.experimental.pallas import tpu as pltpu
from jax.experimental.pallas import tpu_sc as plsc

N = 8192
D = 1024
H = 1024
T = 16
RH = 32

BLK = 256
NBLK = (2 * N + T * BLK) // BLK
NPAD = NBLK * BLK
RB = 512

NW = 32
RPW = NPAD // NW
CH = 80
NCH = RPW // CH


def _router_body(x_ref, w1_ref, b1_ref, w2_ref, b2_ref, ti_ref, tw_ref):
    x = x_ref[...]
    h = jnp.tanh(jnp.dot(x, w1_ref[...], preferred_element_type=jnp.float32)
                 + b1_ref[...])
    s = jnp.dot(h, w2_ref[...], preferred_element_type=jnp.float32) + b2_ref[...]
    i1 = jnp.argmax(s, axis=1).astype(jnp.int32)
    v1 = jnp.max(s, axis=1)
    cols = lax.broadcasted_iota(jnp.int32, s.shape, 1)
    s2 = jnp.where(cols == i1[:, None], -jnp.inf, s)
    i2 = jnp.argmax(s2, axis=1).astype(jnp.int32)
    v2 = jnp.max(s2, axis=1)
    e2 = jnp.exp(v2 - v1)
    denom = 1.0 + e2
    ti_ref[...] = jnp.stack([i1, i2], axis=-1)
    tw_ref[...] = jnp.stack([1.0 / denom, e2 / denom], axis=-1)


def _route(x, router_W1, router_b1, router_W2, router_b2):
    return pl.pallas_call(
        _router_body,
        grid=(N // RB,),
        in_specs=[
            pl.BlockSpec((RB, D), lambda i: (i, 0)),
            pl.BlockSpec((D, RH), lambda i: (0, 0)),
            pl.BlockSpec((1, RH), lambda i: (0, 0)),
            pl.BlockSpec((RH, T), lambda i: (0, 0)),
            pl.BlockSpec((1, T), lambda i: (0, 0)),
        ],
        out_specs=[
            pl.BlockSpec((RB, 2), lambda i: (i, 0)),
            pl.BlockSpec((RB, 2), lambda i: (i, 0)),
        ],
        out_shape=[
            jax.ShapeDtypeStruct((N, 2), jnp.int32),
            jax.ShapeDtypeStruct((N, 2), jnp.float32),
        ],
        compiler_params=pltpu.CompilerParams(
            dimension_semantics=("parallel",)),
    )(x, router_W1, router_b1.reshape(1, RH), router_W2,
      router_b2.reshape(1, T))


@functools.cache
def _make_sc_gather():
    mesh = plsc.VectorSubcoreMesh(core_axis_name="c", subcore_axis_name="s")

    @functools.partial(
        pl.kernel,
        mesh=mesh,
        out_type=jax.ShapeDtypeStruct((NPAD, D // 2), jnp.int32),
        scratch_types=[
            pltpu.VMEM((RPW,), jnp.int32),
            pltpu.VMEM((CH, D // 2), jnp.int32),
            pltpu.VMEM((CH, D // 2), jnp.int32),
            pltpu.SemaphoreType.DMA,
            pltpu.SemaphoreType.DMA,
            pltpu.SemaphoreType.DMA,
            pltpu.SemaphoreType.DMA,
        ],
    )
    def gather_k(x_hbm, idx_hbm, out_hbm, idx_v, rows0, rows1,
                 gsem0, gsem1, wsem0, wsem1):
        wid = lax.axis_index("s") * 2 + lax.axis_index("c")
        base = wid * RPW
        rows = (rows0, rows1)
        gsem = (gsem0, gsem1)
        wsem = (wsem0, wsem1)
        pltpu.sync_copy(idx_hbm.at[pl.ds(base, RPW)], idx_v)
        gathers = [None] * NCH
        writes = [None] * NCH
        gathers[0] = pltpu.async_copy(
            x_hbm.at[idx_v.at[pl.ds(0, CH)]], rows[0], gsem[0])
        for j in range(NCH):
            cur = j % 2
            if j + 1 < NCH:
                nxt = (j + 1) % 2
                if j >= 1:
                    writes[j - 1].wait()
                gathers[j + 1] = pltpu.async_copy(
                    x_hbm.at[idx_v.at[pl.ds((j + 1) * CH, CH)]],
                    rows[nxt], gsem[nxt])
            gathers[j].wait()
            writes[j] = pltpu.async_copy(
                rows[cur], out_hbm.at[pl.ds(base + j * CH, CH)], wsem[cur])
        writes[NCH - 2].wait()
        writes[NCH - 1].wait()

    return gather_k


def _expert_body(be_ref, na_ref, xg_ref, w_ref, b_ref, hw_ref, g_ref, out_ref):
    i = pl.program_id(0)

    @pl.when(i < na_ref[0])
    def _():
        ht = jnp.tanh(
            jnp.dot(xg_ref[...].astype(jnp.float32), w_ref[0],
                    preferred_element_type=jnp.float32) + b_ref[0])
        outs = jnp.sum(ht * hw_ref[0], axis=1)
        out_ref[...] = (outs * g_ref[0, 0, :])[None, None, :]


def _expert_dispatch(xg, tree_W, tree_b, head_W, gates, block_expert, nact):
    def _act(i, be, na):
        return jnp.minimum(i, na[0] - 1)

    grid_spec = pltpu.PrefetchScalarGridSpec(
        num_scalar_prefetch=2,
        grid=(NBLK,),
        in_specs=[
            pl.BlockSpec((BLK, D), lambda i, be, na: (_act(i, be, na), 0)),
            pl.BlockSpec((1, D, H),
                         lambda i, be, na: (be[_act(i, be, na)], 0, 0)),
            pl.BlockSpec((1, 1, H),
                         lambda i, be, na: (be[_act(i, be, na)], 0, 0)),
            pl.BlockSpec((1, 1, H),
                         lambda i, be, na: (be[_act(i, be, na)], 0, 0)),
            pl.BlockSpec((1, 1, BLK),
                         lambda i, be, na: (_act(i, be, na), 0, 0)),
        ],
        out_specs=pl.BlockSpec((1, 1, BLK), lambda i, be, na: (i, 0, 0)),
    )
    return pl.pallas_call(
        _expert_body,
        grid_spec=grid_spec,
        out_shape=jax.ShapeDtypeStruct((NBLK, 1, BLK), jnp.float32),
        compiler_params=pltpu.CompilerParams(
            dimension_semantics=("arbitrary",)),
    )(block_expert, nact, xg, tree_W, tree_b.reshape(T, 1, H),
      head_W.reshape(T, 1, H), gates.reshape(NBLK, 1, BLK))


def kernel(x, router_W1, router_b1, router_W2, router_b2,
           tree_W, tree_b, head_W, head_b, top_k):
    topi, topw = _route(x, router_W1, router_b1, router_W2, router_b2)

    ef = topi.reshape(-1)
    order = jnp.argsort(ef, stable=True).astype(jnp.int32)
    es = ef[order]
    counts = jnp.zeros((T,), jnp.int32).at[ef].add(1)
    padded = ((counts + BLK - 1) // BLK) * BLK
    csum_p = jnp.cumsum(padded)
    gstart = csum_p - padded
    sstart = jnp.cumsum(counts) - counts
    pos = (gstart[es]
           + (jnp.arange(2 * N, dtype=jnp.int32) - sstart[es]))
    tok = (order // 2).astype(jnp.int32)
    ids_padded = jnp.zeros((NPAD,), jnp.int32).at[pos].set(tok)
    gates_padded = jnp.zeros((NPAD,), jnp.float32).at[pos].set(
        topw.reshape(-1)[order])
    dest = jnp.zeros((2 * N,), jnp.int32).at[order].set(pos).reshape(N, 2)

    total = csum_p[T - 1]
    nact = (total // BLK).astype(jnp.int32).reshape(1)
    blk_starts = jnp.arange(NBLK, dtype=jnp.int32) * BLK
    block_expert = jnp.minimum(
        jnp.searchsorted(csum_p, blk_starts, side="right").astype(jnp.int32),
        T - 1)

    x_packed = lax.bitcast_convert_type(
        x.astype(jnp.bfloat16).reshape(N, D // 2, 2), jnp.int32)
    xg_packed = _make_sc_gather()(x_packed, ids_padded)
    xg = lax.bitcast_convert_type(xg_packed, jnp.bfloat16).reshape(NPAD, D)
    contrib = _expert_dispatch(xg, tree_W, tree_b, head_W, gates_padded,
                               block_expert, nact).reshape(NPAD)

    y = (contrib[dest[:, 0]] + contrib[dest[:, 1]]
         + jnp.sum(topw * head_b[topi], axis=1))
    y = y[:, None] + jnp.zeros((N, 1), jnp.float32) * jnp.asarray(
        top_k, dtype=jnp.float32)
    return y


# speedup vs baseline: 2.3575x; 2.3575x over previous
import functools

import jax
import jax.numpy as jnp
from jax import lax
from jax.experimental import pallas as pl
from jax.experimental.pallas import tpu as pltpu
from jax.experimental.pallas import tpu_sc as plsc

N = 8192
D = 1024
H = 1024
T = 16
RH = 32

BLK = 256
NBLK = (2 * N + T * BLK) // BLK
NPAD = NBLK * BLK
RB = 512

NW = 32
DPK = D // 2
CH = 80
NSPLIT = 2
NROWS_SPLIT = NPAD // NSPLIT
NBLK_SPLIT = NBLK // NSPLIT


def _router_body(x_ref, w1_ref, b1_ref, w2_ref, b2_ref, ti_ref, tw_ref,
                 xpk_ref):
    x = x_ref[...]
    au = lax.bitcast_convert_type(x[:, :D // 2], jnp.uint32)
    bu = lax.bitcast_convert_type(x[:, D // 2:], jnp.uint32)
    pk = (((au + 0x8000) >> 16)
          | ((bu + 0x8000) & jnp.uint32(0xffff0000)))
    xpk_ref[...] = lax.bitcast_convert_type(pk, jnp.int32)
    h = jnp.tanh(jnp.dot(x, w1_ref[...], preferred_element_type=jnp.float32)
                 + b1_ref[...])
    s = jnp.dot(h, w2_ref[...], preferred_element_type=jnp.float32) + b2_ref[...]
    i1 = jnp.argmax(s, axis=1).astype(jnp.int32)
    v1 = jnp.max(s, axis=1)
    cols = lax.broadcasted_iota(jnp.int32, s.shape, 1)
    s2 = jnp.where(cols == i1[:, None], -jnp.inf, s)
    i2 = jnp.argmax(s2, axis=1).astype(jnp.int32)
    v2 = jnp.max(s2, axis=1)
    e2 = jnp.exp(v2 - v1)
    denom = 1.0 + e2
    ti_ref[...] = jnp.stack([i1, i2], axis=-1)
    tw_ref[...] = jnp.stack([1.0 / denom, e2 / denom], axis=-1)


def _route(x, router_W1, router_b1, router_W2, router_b2):
    return pl.pallas_call(
        _router_body,
        grid=(N // RB,),
        in_specs=[
            pl.BlockSpec((RB, D), lambda i: (i, 0)),
            pl.BlockSpec((D, RH), lambda i: (0, 0)),
            pl.BlockSpec((1, RH), lambda i: (0, 0)),
            pl.BlockSpec((RH, T), lambda i: (0, 0)),
            pl.BlockSpec((1, T), lambda i: (0, 0)),
        ],
        out_specs=[
            pl.BlockSpec((RB, 2), lambda i: (i, 0)),
            pl.BlockSpec((RB, 2), lambda i: (i, 0)),
            pl.BlockSpec((RB, D // 2), lambda i: (i, 0)),
        ],
        out_shape=[
            jax.ShapeDtypeStruct((N, 2), jnp.int32),
            jax.ShapeDtypeStruct((N, 2), jnp.float32),
            jax.ShapeDtypeStruct((N, D // 2), jnp.int32),
        ],
        compiler_params=pltpu.CompilerParams(
            dimension_semantics=("parallel",)),
    )(x, router_W1, router_b1.reshape(1, RH), router_W2,
      router_b2.reshape(1, T))


@functools.cache
def _make_sc_gather(nrows):
    mesh = plsc.VectorSubcoreMesh(core_axis_name="c", subcore_axis_name="s")
    rpw = nrows // NW
    nch = rpw // CH

    @functools.partial(
        pl.kernel,
        mesh=mesh,
        out_type=jax.ShapeDtypeStruct((nrows, DPK), jnp.int32),
        scratch_types=[
            pltpu.VMEM((rpw,), jnp.int32),
            pltpu.VMEM((CH, DPK), jnp.int32),
            pltpu.VMEM((CH, DPK), jnp.int32),
            pltpu.SemaphoreType.DMA,
            pltpu.SemaphoreType.DMA,
            pltpu.SemaphoreType.DMA,
            pltpu.SemaphoreType.DMA,
        ],
    )
    def gather_k(x_hbm, idx_hbm, out_hbm, idx_v, rows0, rows1,
                 gsem0, gsem1, wsem0, wsem1):
        wid = lax.axis_index("s") * 2 + lax.axis_index("c")
        base = wid * rpw
        rows = (rows0, rows1)
        gsem = (gsem0, gsem1)
        wsem = (wsem0, wsem1)
        pltpu.sync_copy(idx_hbm.at[pl.ds(base, rpw)], idx_v)
        gathers = [None] * nch
        writes = [None] * nch
        gathers[0] = pltpu.async_copy(
            x_hbm.at[idx_v.at[pl.ds(0, CH)]], rows[0], gsem[0])
        for j in range(nch):
            cur = j % 2
            if j + 1 < nch:
                nxt = (j + 1) % 2
                if j >= 1:
                    writes[j - 1].wait()
                gathers[j + 1] = pltpu.async_copy(
                    x_hbm.at[idx_v.at[pl.ds((j + 1) * CH, CH)]],
                    rows[nxt], gsem[nxt])
            gathers[j].wait()
            writes[j] = pltpu.async_copy(
                rows[cur], out_hbm.at[pl.ds(base + j * CH, CH)], wsem[cur])
        writes[nch - 2].wait()
        writes[nch - 1].wait()

    return gather_k


def _expert_body(be_ref, na_ref, xg_ref, w_ref, b_ref, hw_ref, g_ref, out_ref):
    i = pl.program_id(0)

    @pl.when(i < na_ref[0])
    def _():
        xi = lax.bitcast_convert_type(xg_ref[...], jnp.uint32)
        xa = lax.bitcast_convert_type(xi << 16, jnp.float32)
        xb = lax.bitcast_convert_type(xi & jnp.uint32(0xffff0000),
                                      jnp.float32)
        z = (jnp.dot(xa, w_ref[0, :DPK, :], preferred_element_type=jnp.float32)
             + jnp.dot(xb, w_ref[0, DPK:, :],
                       preferred_element_type=jnp.float32))
        ht = jnp.tanh(z + b_ref[0])
        outs = jnp.sum(ht * hw_ref[0], axis=1)
        out_ref[...] = (outs * g_ref[0, 0, :])[None, None, :]


def _expert_dispatch(xg, tree_W, tree_b, head_W, gates, block_expert, nact,
                     nblk):
    def _act(i, be, na):
        return jnp.minimum(i, jnp.maximum(na[0] - 1, 0))

    grid_spec = pltpu.PrefetchScalarGridSpec(
        num_scalar_prefetch=2,
        grid=(nblk,),
        in_specs=[
            pl.BlockSpec((BLK, DPK), lambda i, be, na: (_act(i, be, na), 0)),
            pl.BlockSpec((1, D, H),
                         lambda i, be, na: (be[_act(i, be, na)], 0, 0)),
            pl.BlockSpec((1, 1, H),
                         lambda i, be, na: (be[_act(i, be, na)], 0, 0)),
            pl.BlockSpec((1, 1, H),
                         lambda i, be, na: (be[_act(i, be, na)], 0, 0)),
            pl.BlockSpec((1, 1, BLK),
                         lambda i, be, na: (_act(i, be, na), 0, 0)),
        ],
        out_specs=pl.BlockSpec((1, 1, BLK), lambda i, be, na: (i, 0, 0)),
    )
    return pl.pallas_call(
        _expert_body,
        grid_spec=grid_spec,
        out_shape=jax.ShapeDtypeStruct((nblk, 1, BLK), jnp.float32),
        compiler_params=pltpu.CompilerParams(
            dimension_semantics=("arbitrary",)),
    )(block_expert, nact, xg, tree_W, tree_b.reshape(T, 1, H),
      head_W.reshape(T, 1, H), gates.reshape(nblk, 1, BLK))


def kernel(x, router_W1, router_b1, router_W2, router_b2,
           tree_W, tree_b, head_W, head_b, top_k):
    topi, topw, xpk = _route(x, router_W1, router_b1, router_W2, router_b2)

    ef = topi.reshape(-1)
    order = jnp.argsort(ef, stable=True).astype(jnp.int32)
    es = ef[order]
    counts = jnp.zeros((T,), jnp.int32).at[ef].add(1)
    padded = ((counts + BLK - 1) // BLK) * BLK
    csum_p = jnp.cumsum(padded)
    gstart = csum_p - padded
    sstart = jnp.cumsum(counts) - counts
    pos = (gstart[es]
           + (jnp.arange(2 * N, dtype=jnp.int32) - sstart[es]))
    tok = (order // 2).astype(jnp.int32)
    ids_padded = jnp.zeros((NPAD,), jnp.int32).at[pos].set(tok)
    gates_padded = jnp.zeros((NPAD,), jnp.float32).at[pos].set(
        topw.reshape(-1)[order])
    dest = jnp.zeros((2 * N,), jnp.int32).at[order].set(pos).reshape(N, 2)

    total = csum_p[T - 1]
    nact = (total // BLK).astype(jnp.int32).reshape(1)
    blk_starts = jnp.arange(NBLK, dtype=jnp.int32) * BLK
    block_expert = jnp.minimum(
        jnp.searchsorted(csum_p, blk_starts, side="right").astype(jnp.int32),
        T - 1)

    gather = _make_sc_gather(NROWS_SPLIT)
    contribs = []
    for s in range(NSPLIT):
        xg_s = gather(xpk, lax.dynamic_slice_in_dim(
            ids_padded, s * NROWS_SPLIT, NROWS_SPLIT))
        nact_s = jnp.clip(nact - s * NBLK_SPLIT, 0, NBLK_SPLIT)
        be_s = lax.dynamic_slice_in_dim(block_expert, s * NBLK_SPLIT,
                                        NBLK_SPLIT)
        g_s = lax.dynamic_slice_in_dim(gates_padded, s * NROWS_SPLIT,
                                       NROWS_SPLIT)
        contribs.append(
            _expert_dispatch(xg_s, tree_W, tree_b, head_W, g_s, be_s,
                             nact_s, NBLK_SPLIT).reshape(NROWS_SPLIT))
    contrib = jnp.concatenate(contribs)

    y = (contrib[dest[:, 0]] + contrib[dest[:, 1]]
         + jnp.sum(topw * head_b[topi], axis=1))
    y = y[:, None] + jnp.zeros((N, 1), jnp.float32) * jnp.asarray(
        top_k, dtype=jnp.float32)
    return y
